# baseline (speedup 1.0000x reference)
import functools

import jax
import jax.numpy as jnp
from jax import lax
from jax.experimental import pallas as pl
from jax.experimental.pallas import tpu as pltpu
from jax.experimental.pallas import tpu_sc as plsc

B = 4096
HIST = 50
D = 32
NC, NS = 2, 16
NW = NC * NS
BW = B // NW
IB = BW * HIST
L = 16
NG = BW // L
RT = D // 8


def _transpose_rows(rows_v, blk_v, lanes):
    for g in range(NG):
        rows = lanes + (g * L)
        vals = [plsc.load_gather(rows_v, [rows, lanes * 0 + d])
                for d in range(D)]
        for d in range(D):
            blk_v[d, pl.ds(g * L, L)] = vals[d]


def _transpose_halves(lo_v, hi_v, blk_v, lanes):
    for g in range(NG):
        rows = lanes + (g * L)
        vals = [plsc.load_gather(lo_v, [rows, lanes * 0 + d])
                for d in range(D // 2)]
        vals += [plsc.load_gather(hi_v, [rows, lanes * 0 + d])
                 for d in range(D // 2)]
        for d in range(D):
            blk_v[d, pl.ds(g * L, L)] = vals[d]


def _gather_body(user_id_hbm, items_hbm, user_rows_hbm,
                 item_lo_hbm, item_hi_hbm,
                 user_out_hbm, item_out_hbm,
                 uidx_v, iidx_v, lid0_v, lid1_v, urows_v,
                 lo0_v, lo1_v, hi0_v, hi1_v, blk0_v, blk1_v,
                 gsem0, gsem1, osem0, osem1, usem):
    wid = lax.axis_index("s") * NC + lax.axis_index("c")
    b0 = wid * BW

    pltpu.sync_copy(user_id_hbm.at[pl.ds(b0, BW)], uidx_v)
    pltpu.sync_copy(items_hbm.at[pl.ds(b0 * HIST, IB)], iidx_v)

    lanes = lax.iota(jnp.int32, L)
    lanes50 = lanes * HIST

    lids = (lid0_v, lid1_v)
    los = (lo0_v, lo1_v)
    his = (hi0_v, hi1_v)
    blks = (blk0_v, blk1_v)
    gsems = (gsem0, gsem1)
    osems = (osem0, osem1)

    def prep_fire(l, par):
        for g in range(NG):
            lids[par][pl.ds(g * L, L)] = plsc.load_gather(
                iidx_v, [lanes50 + (g * L * HIST + l)])
        pltpu.async_copy(item_lo_hbm.at[lids[par]], los[par], gsems[par])
        pltpu.async_copy(item_hi_hbm.at[lids[par]], his[par], gsems[par])

    pltpu.async_copy(user_rows_hbm.at[uidx_v], urows_v, usem)
    prep_fire(0, 1)
    pltpu.make_async_copy(user_rows_hbm.at[uidx_v], urows_v, usem).wait()
    _transpose_rows(urows_v, blk0_v, lanes)
    for r in range(RT):
        pltpu.sync_copy(blk0_v.at[pl.ds(8 * r, 8)], user_out_hbm.at[r, wid])

    def step(l, par):
        @pl.when(l + 1 < HIST)
        def _():
            prep_fire(l + 1, 1 - par)
        pltpu.make_async_copy(
            item_lo_hbm.at[lids[par]], los[par], gsems[par]).wait()
        pltpu.make_async_copy(
            item_hi_hbm.at[lids[par]], his[par], gsems[par]).wait()

        @pl.when(l >= 2)
        def _():
            for r in range(RT):
                pltpu.make_async_copy(
                    blks[par].at[pl.ds(8 * r, 8)],
                    item_out_hbm.at[l - 2, r, wid], osems[par]).wait()
        _transpose_halves(los[par], his[par], blks[par], lanes)
        for r in range(RT):
            pltpu.async_copy(blks[par].at[pl.ds(8 * r, 8)],
                             item_out_hbm.at[l, r, wid], osems[par])

    def pair(p, carry):
        l = p * 2
        step(l, 1)
        step(l + 1, 0)
        return carry

    lax.fori_loop(0, HIST // 2, pair, 0)

    for r in range(RT):
        pltpu.make_async_copy(
            blk1_v.at[pl.ds(8 * r, 8)],
            item_out_hbm.at[HIST - 2, r, wid], osem1).wait()
    for r in range(RT):
        pltpu.make_async_copy(
            blk0_v.at[pl.ds(8 * r, 8)],
            item_out_hbm.at[HIST - 1, r, wid], osem0).wait()


_gather = functools.partial(
    pl.kernel,
    out_type=(
        jax.ShapeDtypeStruct((RT, NW, 8, BW), jnp.float32),
        jax.ShapeDtypeStruct((HIST, RT, NW, 8, BW), jnp.float32),
    ),
    mesh=plsc.VectorSubcoreMesh(core_axis_name="c", subcore_axis_name="s",
                                num_cores=NC, num_subcores=NS),
    scratch_types=[
        pltpu.VMEM((BW,), jnp.int32),
        pltpu.VMEM((IB,), jnp.int32),
        pltpu.VMEM((BW,), jnp.int32),
        pltpu.VMEM((BW,), jnp.int32),
        pltpu.VMEM((BW, D), jnp.float32),
        pltpu.VMEM((BW, D // 2), jnp.float32),
        pltpu.VMEM((BW, D // 2), jnp.float32),
        pltpu.VMEM((BW, D // 2), jnp.float32),
        pltpu.VMEM((BW, D // 2), jnp.float32),
        pltpu.VMEM((D, BW), jnp.float32),
        pltpu.VMEM((D, BW), jnp.float32),
        pltpu.SemaphoreType.DMA,
        pltpu.SemaphoreType.DMA,
        pltpu.SemaphoreType.DMA,
        pltpu.SemaphoreType.DMA,
        pltpu.SemaphoreType.DMA,
    ],
    compiler_params=pltpu.CompilerParams(use_tc_tiling_on_sc=False,
                                         needs_layout_passes=False),
)(_gather_body)


def kernel(user_id, items_ids, user_table, item_table):
    items_flat = items_ids.reshape(B * HIST)
    u4, i5 = _gather(user_id, items_flat, user_table,
                     item_table[:, :D // 2], item_table[:, D // 2:])
    user_out = u4.transpose(1, 3, 0, 2).reshape(B, D)
    item_out = i5.transpose(2, 4, 0, 1, 3).reshape(B, HIST, D)
    return user_out, item_out


# speedup vs baseline: 1.9915x; 1.9915x over previous
import functools

import jax
import jax.numpy as jnp
from jax import lax
from jax.experimental import pallas as pl
from jax.experimental.pallas import tpu as pltpu
from jax.experimental.pallas import tpu_sc as plsc

B = 4096
HIST = 50
D = 32
NC, NS = 2, 16
NW = NC * NS
BW = B // NW
IB = BW * HIST
L = 16
NG = BW // L
RT = D // 8


def _transpose_rows(rows_v, blk_v, lanes):
    for g in range(NG):
        rows = lanes + (g * L)
        vals = [plsc.load_gather(rows_v, [rows, lanes * 0 + d])
                for d in range(D)]
        for d in range(D):
            blk_v[d, pl.ds(g * L, L)] = vals[d]


def _gather_body(user_id_hbm, items_hbm, user_rows_hbm, item_rows_hbm,
                 user_out_hbm, item_out_hbm,
                 uidx_v, iidx_v, lid0_v, lid1_v, urows_v,
                 rows0_v, rows1_v, blk0_v, blk1_v,
                 gsem0, gsem1, osem0, osem1, usem):
    wid = lax.axis_index("s") * NC + lax.axis_index("c")
    b0 = wid * BW

    pltpu.sync_copy(user_id_hbm.at[pl.ds(b0, BW)], uidx_v)
    pltpu.sync_copy(items_hbm.at[pl.ds(b0 * HIST, IB)], iidx_v)

    lanes = lax.iota(jnp.int32, L)
    lanes50 = lanes * HIST

    lids = (lid0_v, lid1_v)
    rows = (rows0_v, rows1_v)
    blks = (blk0_v, blk1_v)
    gsems = (gsem0, gsem1)
    osems = (osem0, osem1)

    def prep_fire(l, par):
        for g in range(NG):
            lids[par][pl.ds(g * L, L)] = plsc.load_gather(
                iidx_v, [lanes50 + (g * L * HIST + l)])
        pltpu.async_copy(item_rows_hbm.at[lids[par]], rows[par], gsems[par])

    pltpu.async_copy(user_rows_hbm.at[uidx_v], urows_v, usem)
    prep_fire(0, 1)
    pltpu.make_async_copy(user_rows_hbm.at[uidx_v], urows_v, usem).wait()
    _transpose_rows(urows_v, blk0_v, lanes)
    for r in range(RT):
        pltpu.sync_copy(blk0_v.at[pl.ds(8 * r, 8)], user_out_hbm.at[r, wid])

    def step(l, par):
        @pl.when(l + 1 < HIST)
        def _():
            prep_fire(l + 1, 1 - par)
        pltpu.make_async_copy(
            item_rows_hbm.at[lids[par]], rows[par], gsems[par]).wait()

        @pl.when(l >= 2)
        def _():
            for r in range(RT):
                pltpu.make_async_copy(
                    blks[par].at[pl.ds(8 * r, 8)],
                    item_out_hbm.at[l - 2, r, wid], osems[par]).wait()
        _transpose_rows(rows[par], blks[par], lanes)
        for r in range(RT):
            pltpu.async_copy(blks[par].at[pl.ds(8 * r, 8)],
                             item_out_hbm.at[l, r, wid], osems[par])

    def pair(p, carry):
        l = p * 2
        step(l, 1)
        step(l + 1, 0)
        return carry

    lax.fori_loop(0, HIST // 2, pair, 0)

    for r in range(RT):
        pltpu.make_async_copy(
            blk1_v.at[pl.ds(8 * r, 8)],
            item_out_hbm.at[HIST - 2, r, wid], osem1).wait()
    for r in range(RT):
        pltpu.make_async_copy(
            blk0_v.at[pl.ds(8 * r, 8)],
            item_out_hbm.at[HIST - 1, r, wid], osem0).wait()


_gather = functools.partial(
    pl.kernel,
    out_type=(
        jax.ShapeDtypeStruct((RT, NW, 8, BW), jnp.float32),
        jax.ShapeDtypeStruct((HIST, RT, NW, 8, BW), jnp.float32),
    ),
    mesh=plsc.VectorSubcoreMesh(core_axis_name="c", subcore_axis_name="s",
                                num_cores=NC, num_subcores=NS),
    scratch_types=[
        pltpu.VMEM((BW,), jnp.int32),
        pltpu.VMEM((IB,), jnp.int32),
        pltpu.VMEM((BW,), jnp.int32),
        pltpu.VMEM((BW,), jnp.int32),
        pltpu.VMEM((BW, D), jnp.float32),
        pltpu.VMEM((BW, D), jnp.float32),
        pltpu.VMEM((BW, D), jnp.float32),
        pltpu.VMEM((D, BW), jnp.float32),
        pltpu.VMEM((D, BW), jnp.float32),
        pltpu.SemaphoreType.DMA,
        pltpu.SemaphoreType.DMA,
        pltpu.SemaphoreType.DMA,
        pltpu.SemaphoreType.DMA,
        pltpu.SemaphoreType.DMA,
    ],
    compiler_params=pltpu.CompilerParams(use_tc_tiling_on_sc=False,
                                         needs_layout_passes=False),
)(_gather_body)


def kernel(user_id, items_ids, user_table, item_table):
    items_flat = items_ids.reshape(B * HIST)
    u4, i5 = _gather(user_id, items_flat, user_table, item_table)
    user_out = u4.transpose(1, 3, 0, 2).reshape(B, D)
    item_out = i5.transpose(2, 4, 0, 1, 3).reshape(B, HIST, D)
    return user_out, item_out
